# Initial kernel scaffold; baseline (speedup 1.0000x reference)
import jax
import jax.numpy as jnp
from jax.experimental import pallas as pl


def kernel(q, mu, Wij, dir_ij, pairlist, W1, b1, W2, b2):
    raise NotImplementedError("write your pallas kernel here")


# speedup vs baseline: 7.4543x; 7.4543x over previous
import functools

import jax
import jax.numpy as jnp
from jax.experimental import pallas as pl
from jax.experimental.pallas import tpu as pltpu

N = 10000
E = 320000
F = 128

BN = 2000
BE = 512
NB = E // BE


def _mlp_kernel(q_ref, w1_ref, b1_ref, w2_ref, b2_ref, x_ref):
    h = jnp.dot(q_ref[...], w1_ref[...],
                preferred_element_type=jnp.float32,
                precision=jax.lax.Precision.HIGHEST) + b1_ref[...]
    h = h * jax.nn.sigmoid(h)
    x = jnp.dot(h, w2_ref[...],
                preferred_element_type=jnp.float32,
                precision=jax.lax.Precision.HIGHEST) + b2_ref[...]
    x_ref[...] = x


def _edge_kernel(idxi_ref, idxj_ref, dir_ref, wij_ref, x_ref, mu_ref,
                 outq_ref, outmu_ref):
    @pl.when(pl.program_id(0) == 0)
    def _init():
        outq_ref[...] = jnp.zeros_like(outq_ref)
        outmu_ref[...] = jnp.zeros_like(outmu_ref)

    def body(e, _):
        i = idxi_ref[0, 0, e]
        j = idxj_ref[0, 0, e]
        xj = x_ref[pl.ds(j, 1), :]
        w = wij_ref[pl.ds(e, 1), :]
        xe = w * xj
        dq = xe[:, :F]
        dmuR = xe[:, F:2 * F]
        dmumu = xe[:, 2 * F:]
        muj = mu_ref[pl.ds(j, 1), :]
        d0 = dir_ref[0, 0, e]
        d1 = dir_ref[0, 1, e]
        d2 = dir_ref[0, 2, e]
        dmu = (jnp.concatenate([dmuR * d0, dmuR * d1, dmuR * d2], axis=1)
               + jnp.concatenate([dmumu, dmumu, dmumu], axis=1) * muj)
        outq_ref[pl.ds(i, 1), :] = outq_ref[pl.ds(i, 1), :] + dq
        outmu_ref[pl.ds(i, 1), :] = outmu_ref[pl.ds(i, 1), :] + dmu
        return 0

    jax.lax.fori_loop(0, BE, body, 0)


@jax.jit
def _run(q, mu, Wij, dir_ij, pairlist, W1, b1, W2, b2):
    q2 = q.reshape(N, F)
    mu2 = mu.reshape(N, 3 * F)
    wij2 = Wij.reshape(E, 3 * F)
    idxi = pairlist[0].reshape(NB, 1, BE)
    idxj = pairlist[1].reshape(NB, 1, BE)
    dir3 = dir_ij.T.reshape(3, NB, BE).transpose(1, 0, 2)

    x = pl.pallas_call(
        _mlp_kernel,
        grid=(N // BN,),
        in_specs=[
            pl.BlockSpec((BN, F), lambda s: (s, 0)),
            pl.BlockSpec((F, F), lambda s: (0, 0)),
            pl.BlockSpec((1, F), lambda s: (0, 0)),
            pl.BlockSpec((F, 3 * F), lambda s: (0, 0)),
            pl.BlockSpec((1, 3 * F), lambda s: (0, 0)),
        ],
        out_specs=pl.BlockSpec((BN, 3 * F), lambda s: (s, 0)),
        out_shape=jax.ShapeDtypeStruct((N, 3 * F), jnp.float32),
    )(q2, W1, b1.reshape(1, F), W2, b2.reshape(1, 3 * F))

    outq, outmu = pl.pallas_call(
        _edge_kernel,
        grid=(NB,),
        in_specs=[
            pl.BlockSpec((1, 1, BE), lambda s: (s, 0, 0), memory_space=pltpu.SMEM),
            pl.BlockSpec((1, 1, BE), lambda s: (s, 0, 0), memory_space=pltpu.SMEM),
            pl.BlockSpec((1, 3, BE), lambda s: (s, 0, 0), memory_space=pltpu.SMEM),
            pl.BlockSpec((BE, 3 * F), lambda s: (s, 0)),
            pl.BlockSpec((N, 3 * F), lambda s: (0, 0)),
            pl.BlockSpec((N, 3 * F), lambda s: (0, 0)),
        ],
        out_specs=[
            pl.BlockSpec((N, F), lambda s: (0, 0)),
            pl.BlockSpec((N, 3 * F), lambda s: (0, 0)),
        ],
        out_shape=[
            jax.ShapeDtypeStruct((N, F), jnp.float32),
            jax.ShapeDtypeStruct((N, 3 * F), jnp.float32),
        ],
        compiler_params=pltpu.CompilerParams(
            dimension_semantics=("arbitrary",),
            vmem_limit_bytes=100 * 1024 * 1024,
        ),
    )(idxi, idxj, dir3, wij2, x, mu2)

    return (q + outq.reshape(N, 1, F)), (mu + outmu.reshape(N, 3, F))


def kernel(q, mu, Wij, dir_ij, pairlist, W1, b1, W2, b2):
    return _run(q, mu, Wij, dir_ij, pairlist, W1, b1, W2, b2)


# speedup vs baseline: 11.4543x; 1.5366x over previous
import functools

import jax
import jax.numpy as jnp
from jax import lax
from jax.experimental import pallas as pl
from jax.experimental.pallas import tpu as pltpu
from jax.experimental.pallas import tpu_sc as plsc

N = 10000
E = 320000
F = 128

BN = 2000
BE = 4000
NC = 2
NS = 16
NW = NC * NS
C = 128
NCH = 78
EPW = NCH * C
TAIL0 = NW * EPW
RU = 624
CHK = 104
REM = N - NS * RU
H = 64


def _mlp_kernel(q_ref, mu_ref, w1_ref, b1_ref, w2_ref, b2_ref,
                tq_ref, txlo_ref, txhi_ref, tm01lo_ref, tmb_ref, tm12hi_ref):
    h = jnp.dot(q_ref[...], w1_ref[...],
                preferred_element_type=jnp.float32,
                precision=jax.lax.Precision.HIGHEST) + b1_ref[...]
    h = h * jax.nn.sigmoid(h)
    x = jnp.dot(h, w2_ref[...],
                preferred_element_type=jnp.float32,
                precision=jax.lax.Precision.HIGHEST) + b2_ref[...]
    mu = mu_ref[...]
    tq_ref[...] = x[:, 0:F]
    txlo_ref[...] = jnp.concatenate([x[:, F:F + H], x[:, 2 * F:2 * F + H]],
                                    axis=1)
    txhi_ref[...] = jnp.concatenate([x[:, F + H:2 * F], x[:, 2 * F + H:3 * F]],
                                    axis=1)
    tm01lo_ref[...] = jnp.concatenate([mu[:, 0:H], mu[:, F:F + H]], axis=1)
    tmb_ref[...] = jnp.concatenate([mu[:, 2 * F:2 * F + H], mu[:, H:F]],
                                   axis=1)
    tm12hi_ref[...] = jnp.concatenate([mu[:, F + H:2 * F],
                                       mu[:, 2 * F + H:3 * F]], axis=1)


def _repack_kernel(wij_ref, wrep_ref):
    w = wij_ref[...]
    wrep_ref[...] = jnp.concatenate(
        [w[:, F:F + H], w[:, 2 * F:2 * F + H],
         w[:, F + H:2 * F], w[:, 2 * F + H:3 * F],
         w[:, 0:F]], axis=1)


def _combine_kernel(q_ref, mu_ref, p_ref, qo_ref, mo_ref):
    qo_ref[...] = q_ref[...] + p_ref[0, :, :] + p_ref[4, :, :]
    layout = {(0, 0): (1, 0), (1, 0): (1, 1), (2, 0): (2, 0),
              (0, 1): (2, 1), (1, 1): (3, 0), (2, 1): (3, 1)}
    for (c, hh), (p, blk) in layout.items():
        a = (p_ref[p, :, blk * H:(blk + 1) * H]
             + p_ref[4 + p, :, blk * H:(blk + 1) * H])
        col = c * F + hh * H
        mo_ref[:, col:col + H] = mu_ref[:, col:col + H] + a


def _bcast(vec_ref, e):
    return vec_ref[pl.ds(e, 16)][0]


def _make_sc_kernel():
    mesh = plsc.VectorSubcoreMesh(core_axis_name="c", subcore_axis_name="s")

    @functools.partial(
        pl.kernel,
        out_type=jax.ShapeDtypeStruct((2 * 4, N, F), jnp.float32),
        mesh=mesh,
        scratch_types=[
            pltpu.VMEM((C,), jnp.int32),
            pltpu.VMEM((C,), jnp.int32),
            pltpu.VMEM((C + 16,), jnp.float32),
            pltpu.VMEM((C + 16,), jnp.float32),
            pltpu.VMEM((C, F), jnp.float32),
            pltpu.VMEM((C, F), jnp.float32),
            pltpu.VMEM((C, F), jnp.float32),
            pltpu.VMEM_SHARED((N, F), jnp.float32),
        ],
    )
    def sc_kernel(wrep, tq, txlo, txhi, tm01lo, tmb, tm12hi,
                  idxi_h, idxj_h, d0_h, d1_h, d2_h, zr, out,
                  ji, jj, dbA, dbB, wA, gA, msg, acc):
        cid = lax.axis_index("c")
        sid = lax.axis_index("s")
        wid = cid * NS + sid
        ebase = wid * EPW

        def zero_acc():
            pltpu.sync_copy(zr, msg.at[pl.ds(0, CHK)])
            for k in range(RU // CHK):
                pltpu.sync_copy(msg.at[pl.ds(0, CHK)],
                                acc.at[pl.ds(sid * RU + k * CHK, CHK)])

            @pl.when(sid == NS - 1)
            def _():
                pltpu.sync_copy(msg.at[pl.ds(0, REM)],
                                acc.at[pl.ds(NS * RU, REM)])
            plsc.subcore_barrier()

        def flush(p):
            plsc.subcore_barrier()
            for k in range(RU // CHK):
                r0 = sid * RU + k * CHK
                pltpu.sync_copy(acc.at[pl.ds(r0, CHK)], msg.at[pl.ds(0, CHK)])
                pltpu.sync_copy(msg.at[pl.ds(0, CHK)],
                                out.at[cid * 4 + p, pl.ds(r0, CHK)])

            @pl.when(sid == NS - 1)
            def _():
                pltpu.sync_copy(acc.at[pl.ds(NS * RU, REM)],
                                msg.at[pl.ds(0, REM)])
                pltpu.sync_copy(msg.at[pl.ds(0, REM)],
                                out.at[cid * 4 + p, pl.ds(NS * RU, REM)])

        def load_idx(e0):
            pltpu.sync_copy(idxi_h.at[pl.ds(e0, C)], ji)
            pltpu.sync_copy(idxj_h.at[pl.ds(e0, C)], jj)

        def mul_loop():
            def pe(e, _):
                for h in range(F // 16):
                    sl = pl.ds(h * 16, 16)
                    wA[e, sl] = wA[e, sl] * gA[e, sl]
                return 0
            lax.fori_loop(0, C, pe, 0)

        zero_acc()

        def chunk_q(e0):
            load_idx(e0)
            pltpu.sync_copy(wrep.at[pl.ds(e0, C), pl.ds(2 * F, F)], wA)
            pltpu.sync_copy(tq.at[jj], gA)

            def pe(e, _):
                for h in range(F // 16):
                    sl = pl.ds(h * 16, 16)
                    msg[e, sl] = wA[e, sl] * gA[e, sl]
                return 0
            lax.fori_loop(0, C, pe, 0)
            pltpu.sync_copy(msg, acc.at[ji], add=True)

        lax.fori_loop(0, NCH, lambda t, _: (chunk_q(ebase + t * C), 0)[1], 0)

        @pl.when(wid < 4)
        def _():
            chunk_q(TAIL0 + wid * C)
        flush(0)

        for p, blk, tx, tm, dA_h, dB_h in ((1, 0, txlo, tm01lo, d0_h, d1_h),
                                           (3, 1, txhi, tm12hi, d1_h, d2_h)):
            zero_acc()

            def chunk_pair(e0, blk=blk, tx=tx, tm=tm, dA_h=dA_h, dB_h=dB_h):
                load_idx(e0)
                pltpu.sync_copy(dA_h.at[pl.ds(e0, C)], dbA.at[pl.ds(0, C)])
                pltpu.sync_copy(dB_h.at[pl.ds(e0, C)], dbB.at[pl.ds(0, C)])
                pltpu.sync_copy(wrep.at[pl.ds(e0, C), pl.ds(blk * F, F)], wA)
                pltpu.sync_copy(tx.at[jj], gA)
                mul_loop()
                pltpu.sync_copy(tm.at[jj], gA)

                def pe(e, _):
                    da = _bcast(dbA, e)
                    db = _bcast(dbB, e)
                    for h in range(H // 16):
                        dr = wA[e, pl.ds(h * 16, 16)]
                        dm = wA[e, pl.ds(H + h * 16, 16)]
                        msg[e, pl.ds(h * 16, 16)] = (
                            dr * da + dm * gA[e, pl.ds(h * 16, 16)])
                        msg[e, pl.ds(H + h * 16, 16)] = (
                            dr * db + dm * gA[e, pl.ds(H + h * 16, 16)])
                    return 0
                lax.fori_loop(0, C, pe, 0)
                pltpu.sync_copy(msg, acc.at[ji], add=True)

            lax.fori_loop(0, NCH,
                          lambda t, _: (chunk_pair(ebase + t * C), 0)[1], 0)

            @pl.when(wid < 4)
            def _():
                chunk_pair(TAIL0 + wid * C)
            flush(p)

        zero_acc()

        def chunk_mix(e0):
            load_idx(e0)
            pltpu.sync_copy(d2_h.at[pl.ds(e0, C)], dbA.at[pl.ds(0, C)])
            pltpu.sync_copy(d0_h.at[pl.ds(e0, C)], dbB.at[pl.ds(0, C)])
            pltpu.sync_copy(wrep.at[pl.ds(e0, C), pl.ds(0, F)], wA)
            pltpu.sync_copy(txlo.at[jj], gA)
            mul_loop()
            pltpu.sync_copy(tmb.at[jj], gA)

            def pe1(e, _):
                da = _bcast(dbA, e)
                for h in range(H // 16):
                    dr = wA[e, pl.ds(h * 16, 16)]
                    dm = wA[e, pl.ds(H + h * 16, 16)]
                    msg[e, pl.ds(h * 16, 16)] = (
                        dr * da + dm * gA[e, pl.ds(h * 16, 16)])
                    msg[e, pl.ds(H + h * 16, 16)] = gA[e, pl.ds(H + h * 16, 16)]
                return 0
            lax.fori_loop(0, C, pe1, 0)
            pltpu.sync_copy(wrep.at[pl.ds(e0, C), pl.ds(F, F)], wA)
            pltpu.sync_copy(txhi.at[jj], gA)

            def pe2(e, _):
                db = _bcast(dbB, e)
                for h in range(H // 16):
                    dr = (wA[e, pl.ds(h * 16, 16)]
                          * gA[e, pl.ds(h * 16, 16)])
                    dm = (wA[e, pl.ds(H + h * 16, 16)]
                          * gA[e, pl.ds(H + h * 16, 16)])
                    msg[e, pl.ds(H + h * 16, 16)] = (
                        dr * db + dm * msg[e, pl.ds(H + h * 16, 16)])
                return 0
            lax.fori_loop(0, C, pe2, 0)
            pltpu.sync_copy(msg, acc.at[ji], add=True)

        lax.fori_loop(0, NCH, lambda t, _: (chunk_mix(ebase + t * C), 0)[1], 0)

        @pl.when(wid < 4)
        def _():
            chunk_mix(TAIL0 + wid * C)
        flush(2)

    return sc_kernel


@jax.jit
def _run(q, mu, Wij, dir_ij, pairlist, W1, b1, W2, b2):
    q2 = q.reshape(N, F)
    mu2 = mu.reshape(N, 3 * F)
    wij2 = Wij.reshape(E, 3 * F)
    idxi = pairlist[0]
    idxj = pairlist[1]
    d0 = dir_ij[:, 0] + 0.0
    d1 = dir_ij[:, 1] + 0.0
    d2 = dir_ij[:, 2] + 0.0
    zr = jnp.zeros((CHK, F), jnp.float32)

    tables = pl.pallas_call(
        _mlp_kernel,
        grid=(N // BN,),
        in_specs=[
            pl.BlockSpec((BN, F), lambda s: (s, 0)),
            pl.BlockSpec((BN, 3 * F), lambda s: (s, 0)),
            pl.BlockSpec((F, F), lambda s: (0, 0)),
            pl.BlockSpec((1, F), lambda s: (0, 0)),
            pl.BlockSpec((F, 3 * F), lambda s: (0, 0)),
            pl.BlockSpec((1, 3 * F), lambda s: (0, 0)),
        ],
        out_specs=[pl.BlockSpec((BN, F), lambda s: (s, 0))] * 6,
        out_shape=[jax.ShapeDtypeStruct((N, F), jnp.float32)] * 6,
    )(q2, mu2, W1, b1.reshape(1, F), W2, b2.reshape(1, 3 * F))

    wrep = pl.pallas_call(
        _repack_kernel,
        grid=(E // BE,),
        in_specs=[pl.BlockSpec((BE, 3 * F), lambda s: (s, 0))],
        out_specs=pl.BlockSpec((BE, 3 * F), lambda s: (s, 0)),
        out_shape=jax.ShapeDtypeStruct((E, 3 * F), jnp.float32),
    )(wij2)

    p = _make_sc_kernel()(wrep, *tables, idxi, idxj, d0, d1, d2, zr)

    qo, mo = pl.pallas_call(
        _combine_kernel,
        grid=(N // BN,),
        in_specs=[
            pl.BlockSpec((BN, F), lambda s: (s, 0)),
            pl.BlockSpec((BN, 3 * F), lambda s: (s, 0)),
            pl.BlockSpec((8, BN, F), lambda s: (0, s, 0)),
        ],
        out_specs=[
            pl.BlockSpec((BN, F), lambda s: (s, 0)),
            pl.BlockSpec((BN, 3 * F), lambda s: (s, 0)),
        ],
        out_shape=[
            jax.ShapeDtypeStruct((N, F), jnp.float32),
            jax.ShapeDtypeStruct((N, 3 * F), jnp.float32),
        ],
    )(q2, mu2, p)

    return qo.reshape(N, 1, F), mo.reshape(N, 3, F)


def kernel(q, mu, Wij, dir_ij, pairlist, W1, b1, W2, b2):
    return _run(q, mu, Wij, dir_ij, pairlist, W1, b1, W2, b2)


# speedup vs baseline: 14.9942x; 1.3090x over previous
import functools

import jax
import jax.numpy as jnp
from jax import lax
from jax.experimental import pallas as pl
from jax.experimental.pallas import tpu as pltpu
from jax.experimental.pallas import tpu_sc as plsc

N = 10000
E = 320000
F = 128

BN = 2000
BE = 4000
NC = 2
NS = 16
NW = NC * NS
C = 128
NCH = 78
EPW = NCH * C
TAIL0 = NW * EPW
RU = 624
CHK = 104
REM = N - NS * RU
H = 64


def _mlp_kernel(q_ref, mu_ref, w1_ref, b1_ref, w2_ref, b2_ref,
                tq_ref, txlo_ref, txhi_ref, tm01lo_ref, tmb_ref, tm12hi_ref):
    h = jnp.dot(q_ref[...], w1_ref[...],
                preferred_element_type=jnp.float32,
                precision=jax.lax.Precision.HIGHEST) + b1_ref[...]
    h = h * jax.nn.sigmoid(h)
    x = jnp.dot(h, w2_ref[...],
                preferred_element_type=jnp.float32,
                precision=jax.lax.Precision.HIGHEST) + b2_ref[...]
    mu = mu_ref[...]
    tq_ref[...] = x[:, 0:F]
    txlo_ref[...] = jnp.concatenate([x[:, F:F + H], x[:, 2 * F:2 * F + H]],
                                    axis=1)
    txhi_ref[...] = jnp.concatenate([x[:, F + H:2 * F], x[:, 2 * F + H:3 * F]],
                                    axis=1)
    tm01lo_ref[...] = jnp.concatenate([mu[:, 0:H], mu[:, F:F + H]], axis=1)
    tmb_ref[...] = jnp.concatenate([mu[:, 2 * F:2 * F + H], mu[:, H:F]],
                                   axis=1)
    tm12hi_ref[...] = jnp.concatenate([mu[:, F + H:2 * F],
                                       mu[:, 2 * F + H:3 * F]], axis=1)


def _repack_kernel(wij_ref, wrep_ref):
    w = wij_ref[...]
    wrep_ref[...] = jnp.concatenate(
        [w[:, F:F + H], w[:, 2 * F:2 * F + H],
         w[:, F + H:2 * F], w[:, 2 * F + H:3 * F],
         w[:, 0:F]], axis=1)


def _combine_kernel(q_ref, mu_ref, p_ref, qo_ref, mo_ref):
    qo_ref[...] = q_ref[...] + p_ref[0, :, :] + p_ref[4, :, :]
    layout = {(0, 0): (1, 0), (1, 0): (1, 1), (2, 0): (2, 0),
              (0, 1): (2, 1), (1, 1): (3, 0), (2, 1): (3, 1)}
    for (c, hh), (p, blk) in layout.items():
        a = (p_ref[p, :, blk * H:(blk + 1) * H]
             + p_ref[4 + p, :, blk * H:(blk + 1) * H])
        col = c * F + hh * H
        mo_ref[:, col:col + H] = mu_ref[:, col:col + H] + a


def _bcast(vec_ref, e):
    return vec_ref[pl.ds(e, 16)][0]


def _make_sc_kernel():
    mesh = plsc.VectorSubcoreMesh(core_axis_name="c", subcore_axis_name="s")

    @functools.partial(
        pl.kernel,
        out_type=jax.ShapeDtypeStruct((2 * 4, N, F), jnp.float32),
        mesh=mesh,
        scratch_types=[
            pltpu.VMEM((C,), jnp.int32),
            pltpu.VMEM((C,), jnp.int32),
            pltpu.VMEM((C + 16,), jnp.float32),
            pltpu.VMEM((C + 16,), jnp.float32),
            pltpu.VMEM((C, F), jnp.float32),
            pltpu.VMEM((C, F), jnp.float32),
            pltpu.VMEM((C, F), jnp.float32),
            pltpu.VMEM_SHARED((N, F), jnp.float32),
            pltpu.SemaphoreType.DMA,
        ],
    )
    def sc_kernel(wrep, tq, txlo, txhi, tm01lo, tmb, tm12hi,
                  idxi_h, idxj_h, d0_h, d1_h, d2_h, zr, out,
                  ji, jj, dbA, dbB, wA, gA, msg, acc, sem):
        cid = lax.axis_index("c")
        sid = lax.axis_index("s")
        wid = cid * NS + sid
        ebase = wid * EPW

        def zero_acc():
            pltpu.sync_copy(zr, msg.at[pl.ds(0, CHK)])
            for k in range(RU // CHK):
                pltpu.sync_copy(msg.at[pl.ds(0, CHK)],
                                acc.at[pl.ds(sid * RU + k * CHK, CHK)])

            @pl.when(sid == NS - 1)
            def _():
                pltpu.sync_copy(msg.at[pl.ds(0, REM)],
                                acc.at[pl.ds(NS * RU, REM)])
            plsc.subcore_barrier()

        def flush(p):
            plsc.subcore_barrier()
            for k in range(RU // CHK):
                r0 = sid * RU + k * CHK
                pltpu.sync_copy(acc.at[pl.ds(r0, CHK)], msg.at[pl.ds(0, CHK)])
                pltpu.sync_copy(msg.at[pl.ds(0, CHK)],
                                out.at[cid * 4 + p, pl.ds(r0, CHK)])

            @pl.when(sid == NS - 1)
            def _():
                pltpu.sync_copy(acc.at[pl.ds(NS * RU, REM)],
                                msg.at[pl.ds(0, REM)])
                pltpu.sync_copy(msg.at[pl.ds(0, REM)],
                                out.at[cid * 4 + p, pl.ds(NS * RU, REM)])

        def load_idx(e0):
            pltpu.sync_copy(idxi_h.at[pl.ds(e0, C)], ji)
            pltpu.sync_copy(idxj_h.at[pl.ds(e0, C)], jj)

        zero_acc()

        def chunk_q(e0):
            c1 = pltpu.async_copy(idxi_h.at[pl.ds(e0, C)], ji, sem)
            c2 = pltpu.async_copy(idxj_h.at[pl.ds(e0, C)], jj, sem)
            c3 = pltpu.async_copy(wrep.at[pl.ds(e0, C), pl.ds(2 * F, F)],
                                  wA, sem)
            c1.wait(); c2.wait(); c3.wait()
            pltpu.sync_copy(tq.at[jj], gA)

            def pe(e, _):
                for h in range(F // 16):
                    sl = pl.ds(h * 16, 16)
                    msg[e, sl] = wA[e, sl] * gA[e, sl]
                return 0
            lax.fori_loop(0, C, pe, 0)
            pltpu.sync_copy(msg, acc.at[ji], add=True)

        lax.fori_loop(0, NCH, lambda t, _: (chunk_q(ebase + t * C), 0)[1], 0)

        @pl.when(wid < 4)
        def _():
            chunk_q(TAIL0 + wid * C)
        flush(0)

        for p, blk, tx, tm, dA_h, dB_h in ((1, 0, txlo, tm01lo, d0_h, d1_h),
                                           (3, 1, txhi, tm12hi, d1_h, d2_h)):
            zero_acc()

            def chunk_pair(e0, blk=blk, tx=tx, tm=tm, dA_h=dA_h, dB_h=dB_h):
                c1 = pltpu.async_copy(idxi_h.at[pl.ds(e0, C)], ji, sem)
                c2 = pltpu.async_copy(idxj_h.at[pl.ds(e0, C)], jj, sem)
                c3 = pltpu.async_copy(dA_h.at[pl.ds(e0, C)],
                                      dbA.at[pl.ds(0, C)], sem)
                c4 = pltpu.async_copy(dB_h.at[pl.ds(e0, C)],
                                      dbB.at[pl.ds(0, C)], sem)
                c5 = pltpu.async_copy(wrep.at[pl.ds(e0, C), pl.ds(blk * F, F)],
                                      wA, sem)
                c1.wait(); c2.wait(); c3.wait(); c4.wait(); c5.wait()
                g1 = pltpu.async_copy(tx.at[jj], gA, sem)
                g2 = pltpu.async_copy(tm.at[jj], msg, sem)
                g1.wait(); g2.wait()

                def pe(e, _):
                    da = _bcast(dbA, e)
                    db = _bcast(dbB, e)
                    for h in range(H // 16):
                        dr = (wA[e, pl.ds(h * 16, 16)]
                              * gA[e, pl.ds(h * 16, 16)])
                        dm = (wA[e, pl.ds(H + h * 16, 16)]
                              * gA[e, pl.ds(H + h * 16, 16)])
                        msg[e, pl.ds(h * 16, 16)] = (
                            dr * da + dm * msg[e, pl.ds(h * 16, 16)])
                        msg[e, pl.ds(H + h * 16, 16)] = (
                            dr * db + dm * msg[e, pl.ds(H + h * 16, 16)])
                    return 0
                lax.fori_loop(0, C, pe, 0)
                pltpu.sync_copy(msg, acc.at[ji], add=True)

            lax.fori_loop(0, NCH,
                          lambda t, _: (chunk_pair(ebase + t * C), 0)[1], 0)

            @pl.when(wid < 4)
            def _():
                chunk_pair(TAIL0 + wid * C)
            flush(p)

        zero_acc()

        def chunk_mix(e0):
            c1 = pltpu.async_copy(idxi_h.at[pl.ds(e0, C)], ji, sem)
            c2 = pltpu.async_copy(idxj_h.at[pl.ds(e0, C)], jj, sem)
            c3 = pltpu.async_copy(d2_h.at[pl.ds(e0, C)],
                                  dbA.at[pl.ds(0, C)], sem)
            c4 = pltpu.async_copy(d0_h.at[pl.ds(e0, C)],
                                  dbB.at[pl.ds(0, C)], sem)
            c5 = pltpu.async_copy(wrep.at[pl.ds(e0, C), pl.ds(0, F)], wA, sem)
            c1.wait(); c2.wait(); c3.wait(); c4.wait(); c5.wait()
            g1 = pltpu.async_copy(txlo.at[jj], gA, sem)
            g2 = pltpu.async_copy(tmb.at[jj], msg, sem)
            g1.wait(); g2.wait()

            def pe1(e, _):
                da = _bcast(dbA, e)
                for h in range(H // 16):
                    dr = (wA[e, pl.ds(h * 16, 16)]
                          * gA[e, pl.ds(h * 16, 16)])
                    dm = (wA[e, pl.ds(H + h * 16, 16)]
                          * gA[e, pl.ds(H + h * 16, 16)])
                    msg[e, pl.ds(h * 16, 16)] = (
                        dr * da + dm * msg[e, pl.ds(h * 16, 16)])
                return 0
            lax.fori_loop(0, C, pe1, 0)
            g3 = pltpu.async_copy(wrep.at[pl.ds(e0, C), pl.ds(F, F)], wA, sem)
            g4 = pltpu.async_copy(txhi.at[jj], gA, sem)
            g3.wait(); g4.wait()

            def pe2(e, _):
                db = _bcast(dbB, e)
                for h in range(H // 16):
                    dr = (wA[e, pl.ds(h * 16, 16)]
                          * gA[e, pl.ds(h * 16, 16)])
                    dm = (wA[e, pl.ds(H + h * 16, 16)]
                          * gA[e, pl.ds(H + h * 16, 16)])
                    msg[e, pl.ds(H + h * 16, 16)] = (
                        dr * db + dm * msg[e, pl.ds(H + h * 16, 16)])
                return 0
            lax.fori_loop(0, C, pe2, 0)
            pltpu.sync_copy(msg, acc.at[ji], add=True)

        lax.fori_loop(0, NCH, lambda t, _: (chunk_mix(ebase + t * C), 0)[1], 0)

        @pl.when(wid < 4)
        def _():
            chunk_mix(TAIL0 + wid * C)
        flush(2)

    return sc_kernel


@jax.jit
def _run(q, mu, Wij, dir_ij, pairlist, W1, b1, W2, b2):
    q2 = q.reshape(N, F)
    mu2 = mu.reshape(N, 3 * F)
    wij2 = Wij.reshape(E, 3 * F)
    idxi = pairlist[0]
    idxj = pairlist[1]
    d0 = dir_ij[:, 0] + 0.0
    d1 = dir_ij[:, 1] + 0.0
    d2 = dir_ij[:, 2] + 0.0
    zr = jnp.zeros((CHK, F), jnp.float32)

    tables = pl.pallas_call(
        _mlp_kernel,
        grid=(N // BN,),
        in_specs=[
            pl.BlockSpec((BN, F), lambda s: (s, 0)),
            pl.BlockSpec((BN, 3 * F), lambda s: (s, 0)),
            pl.BlockSpec((F, F), lambda s: (0, 0)),
            pl.BlockSpec((1, F), lambda s: (0, 0)),
            pl.BlockSpec((F, 3 * F), lambda s: (0, 0)),
            pl.BlockSpec((1, 3 * F), lambda s: (0, 0)),
        ],
        out_specs=[pl.BlockSpec((BN, F), lambda s: (s, 0))] * 6,
        out_shape=[jax.ShapeDtypeStruct((N, F), jnp.float32)] * 6,
    )(q2, mu2, W1, b1.reshape(1, F), W2, b2.reshape(1, 3 * F))

    wrep = pl.pallas_call(
        _repack_kernel,
        grid=(E // BE,),
        in_specs=[pl.BlockSpec((BE, 3 * F), lambda s: (s, 0))],
        out_specs=pl.BlockSpec((BE, 3 * F), lambda s: (s, 0)),
        out_shape=jax.ShapeDtypeStruct((E, 3 * F), jnp.float32),
    )(wij2)

    p = _make_sc_kernel()(wrep, *tables, idxi, idxj, d0, d1, d2, zr)

    qo, mo = pl.pallas_call(
        _combine_kernel,
        grid=(N // BN,),
        in_specs=[
            pl.BlockSpec((BN, F), lambda s: (s, 0)),
            pl.BlockSpec((BN, 3 * F), lambda s: (s, 0)),
            pl.BlockSpec((8, BN, F), lambda s: (0, s, 0)),
        ],
        out_specs=[
            pl.BlockSpec((BN, F), lambda s: (s, 0)),
            pl.BlockSpec((BN, 3 * F), lambda s: (s, 0)),
        ],
        out_shape=[
            jax.ShapeDtypeStruct((N, F), jnp.float32),
            jax.ShapeDtypeStruct((N, 3 * F), jnp.float32),
        ],
    )(q2, mu2, p)

    return qo.reshape(N, 1, F), mo.reshape(N, 3, F)


def kernel(q, mu, Wij, dir_ij, pairlist, W1, b1, W2, b2):
    return _run(q, mu, Wij, dir_ij, pairlist, W1, b1, W2, b2)
